# Initial kernel scaffold; baseline (speedup 1.0000x reference)
import jax
import jax.numpy as jnp
from jax.experimental import pallas as pl


def kernel(local_preds, edge_index, W1, W2):
    raise NotImplementedError("write your pallas kernel here")


# speedup vs baseline: 22.0251x; 22.0251x over previous
import jax
import jax.numpy as jnp
from jax import lax
from jax.experimental import pallas as pl
from jax.experimental.pallas import tpu as pltpu
from jax.experimental.pallas import tpu_sc as plsc

N = 10000
E = 320000
IN_FEATS = 128
N_HIDDEN = 64
C = 16
ALPHA = 0.1
NITER = 10

NS = 16
NP = 10240
RPT = NP // NS
EP = 327680
EPT = EP // NS
CH = 2048
NCHUNK = EPT // CH

_MESH = dict(
    mesh=plsc.VectorSubcoreMesh(
        core_axis_name="c", subcore_axis_name="s", num_cores=1, num_subcores=NS
    ),
    compiler_params=pltpu.CompilerParams(use_tc_tiling_on_sc=False),
)


def _degree_body(row_hbm, ones_hbm, deg_out, acc_sh, idx_v, ones_v, sem):
    w = lax.axis_index("s")
    base_r = pl.multiple_of(w * RPT, RPT)
    pltpu.sync_copy(ones_hbm, ones_v)
    pltpu.sync_copy(ones_v.at[pl.ds(0, RPT)], acc_sh.at[pl.ds(base_r, RPT)])
    plsc.subcore_barrier()

    def chunk(j, carry):
        e0 = pl.multiple_of(w * EPT + j * CH, CH)
        pltpu.sync_copy(row_hbm.at[pl.ds(e0, CH)], idx_v)
        pltpu.sync_copy(ones_v, acc_sh.at[idx_v], add=True)
        return carry

    lax.fori_loop(0, NCHUNK, chunk, 0)
    plsc.subcore_barrier()
    pltpu.sync_copy(acc_sh.at[pl.ds(base_r, RPT)], ones_v.at[pl.ds(0, RPT)])
    pltpu.sync_copy(ones_v.at[pl.ds(0, RPT)], deg_out.at[pl.ds(base_r, RPT)])


_BLK = 1024


def _dense_body(x_ref, w1_ref, w2_ref, deg_ref, q0_ref, sa_ref, sb_ref,
                sal_ref, sbl_ref):
    h = jnp.tanh(jnp.dot(x_ref[...], w1_ref[...],
                         preferred_element_type=jnp.float32))
    loc = jnp.dot(h, w2_ref[...], preferred_element_type=jnp.float32)
    dinv = lax.rsqrt(deg_ref[...])
    q0 = dinv * loc
    q0_ref[...] = q0
    sa_ref[...] = 0.9 * dinv * dinv
    sb_ref[...] = ALPHA * q0
    sal_ref[...] = 0.9 * dinv
    sbl_ref[...] = ALPHA * loc


def _dense_stage(x, w1, w2, deg_b):
    outs = [jax.ShapeDtypeStruct((NP, C), jnp.float32)] * 5
    return pl.pallas_call(
        _dense_body,
        grid=(NP // _BLK,),
        in_specs=[
            pl.BlockSpec((_BLK, IN_FEATS), lambda i: (i, 0)),
            pl.BlockSpec((IN_FEATS, N_HIDDEN), lambda i: (0, 0)),
            pl.BlockSpec((N_HIDDEN, C), lambda i: (0, 0)),
            pl.BlockSpec((_BLK, C), lambda i: (i, 0)),
        ],
        out_specs=[pl.BlockSpec((_BLK, C), lambda i: (i, 0))] * 5,
        out_shape=outs,
    )(x, w1, w2, deg_b)


def _prop_body(q_hbm, col_hbm, row_hbm, sa_hbm, sb_hbm, qn_hbm,
               acc_sh, cidx_v, ridx_v, msg_v, a_v, b_v, c_v, sem):
    w = lax.axis_index("s")
    base_r = pl.multiple_of(w * RPT, RPT)
    pltpu.sync_copy(q_hbm.at[pl.ds(base_r, RPT)], a_v)
    pltpu.sync_copy(a_v, acc_sh.at[pl.ds(base_r, RPT)])
    plsc.subcore_barrier()

    def chunk(j, carry):
        e0 = pl.multiple_of(w * EPT + j * CH, CH)
        pltpu.sync_copy(col_hbm.at[pl.ds(e0, CH)], cidx_v)
        pltpu.async_copy(q_hbm.at[cidx_v], msg_v, sem).wait()
        pltpu.sync_copy(row_hbm.at[pl.ds(e0, CH)], ridx_v)
        pltpu.sync_copy(msg_v, acc_sh.at[ridx_v], add=True)
        return carry

    lax.fori_loop(0, NCHUNK, chunk, 0)
    plsc.subcore_barrier()

    pltpu.sync_copy(acc_sh.at[pl.ds(base_r, RPT)], a_v)
    pltpu.sync_copy(sa_hbm.at[pl.ds(base_r, RPT)], b_v)
    pltpu.sync_copy(sb_hbm.at[pl.ds(base_r, RPT)], c_v)

    def row(i, carry):
        a_v[i, :] = a_v[i, :] * b_v[i, :] + c_v[i, :]
        return carry

    lax.fori_loop(0, RPT, row, 0)
    pltpu.sync_copy(a_v, qn_hbm.at[pl.ds(base_r, RPT)])


def _build(interpret=False):
    degree_kernel = pl.kernel(
        _degree_body,
        out_type=jax.ShapeDtypeStruct((NP, C), jnp.float32),
        scratch_types=[
            pltpu.VMEM_SHARED((NP, C), jnp.float32),
            pltpu.VMEM((CH,), jnp.int32),
            pltpu.VMEM((CH, C), jnp.float32),
            pltpu.SemaphoreType.DMA,
        ],
        interpret=interpret,
        **_MESH,
    )
    prop_kernel = pl.kernel(
        _prop_body,
        out_type=jax.ShapeDtypeStruct((NP, C), jnp.float32),
        scratch_types=[
            pltpu.VMEM_SHARED((NP, C), jnp.float32),
            pltpu.VMEM((CH,), jnp.int32),
            pltpu.VMEM((CH,), jnp.int32),
            pltpu.VMEM((CH, C), jnp.float32),
            pltpu.VMEM((RPT, C), jnp.float32),
            pltpu.VMEM((RPT, C), jnp.float32),
            pltpu.VMEM((RPT, C), jnp.float32),
            pltpu.SemaphoreType.DMA,
        ],
        interpret=interpret,
        **_MESH,
    )
    return degree_kernel, prop_kernel


_degree_kernel, _prop_kernel = _build()


def kernel(local_preds, edge_index, W1, W2):
    npad = EP - E
    row = jnp.concatenate(
        [edge_index[0], N + (jnp.arange(npad, dtype=jnp.int32) % (NP - N))])
    col = jnp.concatenate([edge_index[1], jnp.zeros(npad, dtype=jnp.int32)])
    x = jnp.pad(local_preds, ((0, NP - N), (0, 0)))
    ones_chunk = jnp.ones((CH, C), dtype=jnp.float32)

    deg_b = _degree_kernel(row, ones_chunk)
    q0, sa, sb, sal, sbl = _dense_stage(x, W1, W2, deg_b)

    q = q0
    for _ in range(NITER - 1):
        q = _prop_kernel(q, col, row, sa, sb)
    preds = _prop_kernel(q, col, row, sal, sbl)
    return preds[:N]


# speedup vs baseline: 27.1329x; 1.2319x over previous
import jax
import jax.numpy as jnp
from jax import lax
from jax.experimental import pallas as pl
from jax.experimental.pallas import tpu as pltpu
from jax.experimental.pallas import tpu_sc as plsc

N = 10000
E = 320000
IN_FEATS = 128
N_HIDDEN = 64
C = 16
ALPHA = 0.1
NITER = 10

NS = 16
NP = 10240
RPT = NP // NS
EP = 327680
EPT = EP // NS
CH = 2048
NCHUNK = EPT // CH

_MESH = dict(
    mesh=plsc.VectorSubcoreMesh(
        core_axis_name="c", subcore_axis_name="s", num_cores=1, num_subcores=NS
    ),
    compiler_params=pltpu.CompilerParams(use_tc_tiling_on_sc=False),
)


def _degree_body(row_hbm, ones_hbm, deg_out, acc_sh, idx_v, ones_v, sem):
    w = lax.axis_index("s")
    base_r = pl.multiple_of(w * RPT, RPT)
    pltpu.sync_copy(ones_hbm, ones_v)
    pltpu.sync_copy(ones_v.at[pl.ds(0, RPT)], acc_sh.at[pl.ds(base_r, RPT)])
    plsc.subcore_barrier()

    def chunk(j, carry):
        e0 = pl.multiple_of(w * EPT + j * CH, CH)
        pltpu.sync_copy(row_hbm.at[pl.ds(e0, CH)], idx_v)
        pltpu.sync_copy(ones_v, acc_sh.at[idx_v], add=True)
        return carry

    lax.fori_loop(0, NCHUNK, chunk, 0)
    plsc.subcore_barrier()
    pltpu.sync_copy(acc_sh.at[pl.ds(base_r, RPT)], ones_v.at[pl.ds(0, RPT)])
    pltpu.sync_copy(ones_v.at[pl.ds(0, RPT)], deg_out.at[pl.ds(base_r, RPT)])


_BLK = 1024


def _dense_body(x_ref, w1_ref, w2_ref, deg_ref, q0_ref, sa_ref, sb_ref,
                sal_ref, sbl_ref):
    h = jnp.tanh(jnp.dot(x_ref[...], w1_ref[...],
                         preferred_element_type=jnp.float32))
    loc = jnp.dot(h, w2_ref[...], preferred_element_type=jnp.float32)
    dinv = lax.rsqrt(deg_ref[...])
    q0 = dinv * loc
    q0_ref[...] = q0
    sa_ref[...] = 0.9 * dinv * dinv
    sb_ref[...] = ALPHA * q0
    sal_ref[...] = 0.9 * dinv
    sbl_ref[...] = ALPHA * loc


def _dense_stage(x, w1, w2, deg_b):
    outs = [jax.ShapeDtypeStruct((NP, C), jnp.float32)] * 5
    return pl.pallas_call(
        _dense_body,
        grid=(NP // _BLK,),
        in_specs=[
            pl.BlockSpec((_BLK, IN_FEATS), lambda i: (i, 0)),
            pl.BlockSpec((IN_FEATS, N_HIDDEN), lambda i: (0, 0)),
            pl.BlockSpec((N_HIDDEN, C), lambda i: (0, 0)),
            pl.BlockSpec((_BLK, C), lambda i: (i, 0)),
        ],
        out_specs=[pl.BlockSpec((_BLK, C), lambda i: (i, 0))] * 5,
        out_shape=outs,
    )(x, w1, w2, deg_b)


def _prop_body(q0_hbm, col2_hbm, row2_hbm, sa_hbm, sb_hbm, sal_hbm, sbl_hbm,
               out_hbm, tmp_hbm,
               acc_sh, cidx_a, ridx_a, msg0, msg1, c_v, sem0, sem1):
    w = lax.axis_index("s")
    base_r = pl.multiple_of(w * RPT, RPT)
    rsl = pl.ds(base_r, RPT)
    pltpu.sync_copy(col2_hbm.at[pl.ds(w * NCHUNK, NCHUNK)], cidx_a)
    pltpu.sync_copy(row2_hbm.at[pl.ds(w * NCHUNK, NCHUNK)], ridx_a)

    msgs = (msg0, msg1)
    sems = (sem0, sem1)

    for t in range(NITER):
        src = q0_hbm if t == 0 else (out_hbm if t % 2 == 0 else tmp_hbm)
        dst = tmp_hbm if t % 2 == 0 else out_hbm
        sa = sa_hbm if t < NITER - 1 else sal_hbm
        sb = sb_hbm if t < NITER - 1 else sbl_hbm
        pltpu.sync_copy(src.at[rsl], acc_sh.at[rsl])
        plsc.subcore_barrier()

        cps = [None, None]
        cps[0] = pltpu.async_copy(src.at[cidx_a.at[0]], msg0, sem0)
        for j in range(NCHUNK):
            b = j % 2
            cps[b].wait()
            if j + 1 < NCHUNK:
                nb = (j + 1) % 2
                cps[nb] = pltpu.async_copy(
                    src.at[cidx_a.at[j + 1]], msgs[nb], sems[nb])
            pltpu.sync_copy(msgs[b], acc_sh.at[ridx_a.at[j]], add=True)
        plsc.subcore_barrier()

        pltpu.sync_copy(acc_sh.at[rsl], c_v)
        pltpu.sync_copy(sa.at[rsl], msg0.at[pl.ds(0, RPT)])
        pltpu.sync_copy(sb.at[rsl], msg1.at[pl.ds(0, RPT)])

        def row(i, carry):
            c_v[i, :] = c_v[i, :] * msg0[i, :] + msg1[i, :]
            return carry

        lax.fori_loop(0, RPT, row, 0)
        pltpu.sync_copy(c_v, dst.at[rsl])
        plsc.subcore_barrier()


def _build(interpret=False):
    degree_kernel = pl.kernel(
        _degree_body,
        out_type=jax.ShapeDtypeStruct((NP, C), jnp.float32),
        scratch_types=[
            pltpu.VMEM_SHARED((NP, C), jnp.float32),
            pltpu.VMEM((CH,), jnp.int32),
            pltpu.VMEM((CH, C), jnp.float32),
            pltpu.SemaphoreType.DMA,
        ],
        interpret=interpret,
        **_MESH,
    )
    prop_kernel = pl.kernel(
        _prop_body,
        out_type=(
            jax.ShapeDtypeStruct((NP, C), jnp.float32),
            jax.ShapeDtypeStruct((NP, C), jnp.float32),
        ),
        scratch_types=[
            pltpu.VMEM_SHARED((NP, C), jnp.float32),
            pltpu.VMEM((NCHUNK, CH), jnp.int32),
            pltpu.VMEM((NCHUNK, CH), jnp.int32),
            pltpu.VMEM((CH, C), jnp.float32),
            pltpu.VMEM((CH, C), jnp.float32),
            pltpu.VMEM((RPT, C), jnp.float32),
            pltpu.SemaphoreType.DMA,
            pltpu.SemaphoreType.DMA,
        ],
        interpret=interpret,
        **_MESH,
    )
    return degree_kernel, prop_kernel


_degree_kernel, _prop_kernel = _build()


def kernel(local_preds, edge_index, W1, W2):
    npad = EP - E
    row = jnp.concatenate(
        [edge_index[0], N + (jnp.arange(npad, dtype=jnp.int32) % (NP - N))])
    col = jnp.concatenate([edge_index[1], jnp.zeros(npad, dtype=jnp.int32)])
    x = jnp.pad(local_preds, ((0, NP - N), (0, 0)))
    ones_chunk = jnp.ones((CH, C), dtype=jnp.float32)

    deg_b = _degree_kernel(row, ones_chunk)
    q0, sa, sb, sal, sbl = _dense_stage(x, W1, W2, deg_b)

    col2 = col.reshape(NS * NCHUNK, CH)
    row2 = row.reshape(NS * NCHUNK, CH)
    preds, _ = _prop_kernel(q0, col2, row2, sa, sb, sal, sbl)
    return preds[:N]


# speedup vs baseline: 48.2822x; 1.7795x over previous
import jax
import jax.numpy as jnp
from jax import lax
from jax.experimental import pallas as pl
from jax.experimental.pallas import tpu as pltpu
from jax.experimental.pallas import tpu_sc as plsc

N = 10000
E = 320000
IN_FEATS = 128
N_HIDDEN = 64
C = 16
ALPHA = 0.1
NITER = 10

NS = 16
NP = 10240
RPT = NP // NS
EP = 327680
EPT = EP // NS
CH = 1024
NCHUNK = EPT // CH

_MESH = dict(
    mesh=plsc.VectorSubcoreMesh(
        core_axis_name="c", subcore_axis_name="s", num_cores=1, num_subcores=NS
    ),
    compiler_params=pltpu.CompilerParams(use_tc_tiling_on_sc=False),
)


def _degree_body(row_hbm, ones_hbm, deg_out, acc_sh, idx_v, ones_v, sem):
    w = lax.axis_index("s")
    base_r = pl.multiple_of(w * RPT, RPT)
    pltpu.sync_copy(ones_hbm, ones_v)
    pltpu.sync_copy(ones_v.at[pl.ds(0, RPT)], acc_sh.at[pl.ds(base_r, RPT)])
    plsc.subcore_barrier()

    def chunk(j, carry):
        e0 = pl.multiple_of(w * EPT + j * CH, CH)
        pltpu.sync_copy(row_hbm.at[pl.ds(e0, CH)], idx_v)
        pltpu.sync_copy(ones_v, acc_sh.at[idx_v], add=True)
        return carry

    lax.fori_loop(0, NCHUNK, chunk, 0)
    plsc.subcore_barrier()
    pltpu.sync_copy(acc_sh.at[pl.ds(base_r, RPT)], ones_v.at[pl.ds(0, RPT)])
    pltpu.sync_copy(ones_v.at[pl.ds(0, RPT)], deg_out.at[pl.ds(base_r, RPT)])


_BLK = 1024


def _dense_body(x_ref, w1_ref, w2_ref, deg_ref, q0_ref, sa_ref, sb_ref,
                sal_ref, sbl_ref):
    h = jnp.tanh(jnp.dot(x_ref[...], w1_ref[...],
                         preferred_element_type=jnp.float32))
    loc = jnp.dot(h, w2_ref[...], preferred_element_type=jnp.float32)
    dinv = lax.rsqrt(deg_ref[...])
    q0 = dinv * loc
    q0_ref[...] = q0
    sa_ref[...] = 0.9 * dinv * dinv
    sb_ref[...] = ALPHA * q0
    sal_ref[...] = 0.9 * dinv
    sbl_ref[...] = ALPHA * loc


def _dense_stage(x, w1, w2, deg_b):
    outs = [jax.ShapeDtypeStruct((NP, C), jnp.float32)] * 5
    return pl.pallas_call(
        _dense_body,
        grid=(NP // _BLK,),
        in_specs=[
            pl.BlockSpec((_BLK, IN_FEATS), lambda i: (i, 0)),
            pl.BlockSpec((IN_FEATS, N_HIDDEN), lambda i: (0, 0)),
            pl.BlockSpec((N_HIDDEN, C), lambda i: (0, 0)),
            pl.BlockSpec((_BLK, C), lambda i: (i, 0)),
        ],
        out_specs=[pl.BlockSpec((_BLK, C), lambda i: (i, 0))] * 5,
        out_shape=outs,
    )(x, w1, w2, deg_b)


def _prop_body(q0_hbm, col2_hbm, row2_hbm, sa_hbm, sb_hbm, sal_hbm, sbl_hbm,
               out_hbm,
               q_sh, acc_sh, cidx_a, ridx_a, msg0, msg1, c_v, sem0, sem1):
    w = lax.axis_index("s")
    base_r = pl.multiple_of(w * RPT, RPT)
    rsl = pl.ds(base_r, RPT)
    pltpu.sync_copy(col2_hbm.at[pl.ds(w * NCHUNK, NCHUNK)], cidx_a)
    pltpu.sync_copy(row2_hbm.at[pl.ds(w * NCHUNK, NCHUNK)], ridx_a)
    bufs = (q_sh, acc_sh)
    pltpu.sync_copy(q0_hbm.at[rsl], q_sh.at[rsl])
    pltpu.sync_copy(q0_hbm.at[rsl], acc_sh.at[rsl])
    plsc.subcore_barrier()

    msgs = (msg0, msg1)
    sems = (sem0, sem1)

    for t in range(NITER):
        S = bufs[t % 2]
        ACC = bufs[(t + 1) % 2]
        sa = sa_hbm if t < NITER - 1 else sal_hbm
        sb = sb_hbm if t < NITER - 1 else sbl_hbm

        cps = [None, None]
        cps[0] = pltpu.async_copy(S.at[cidx_a.at[0]], msg0, sem0)
        for j in range(NCHUNK):
            b = j % 2
            cps[b].wait()
            if j + 1 < NCHUNK:
                nb = (j + 1) % 2
                cps[nb] = pltpu.async_copy(
                    S.at[cidx_a.at[j + 1]], msgs[nb], sems[nb])
            pltpu.sync_copy(msgs[b], ACC.at[ridx_a.at[j]], add=True)
        plsc.subcore_barrier()

        pltpu.sync_copy(ACC.at[rsl], c_v)
        pltpu.sync_copy(sa.at[rsl], msg0.at[pl.ds(0, RPT)])
        pltpu.sync_copy(sb.at[rsl], msg1.at[pl.ds(0, RPT)])

        def row(i, carry):
            c_v[i, :] = c_v[i, :] * msg0[i, :] + msg1[i, :]
            return carry

        lax.fori_loop(0, RPT, row, 0)
        if t < NITER - 1:
            pltpu.sync_copy(c_v, ACC.at[rsl])
            pltpu.sync_copy(c_v, S.at[rsl])
        else:
            pltpu.sync_copy(c_v, out_hbm.at[rsl])
        plsc.subcore_barrier()


def _build(interpret=False):
    degree_kernel = pl.kernel(
        _degree_body,
        out_type=jax.ShapeDtypeStruct((NP, C), jnp.float32),
        scratch_types=[
            pltpu.VMEM_SHARED((NP, C), jnp.float32),
            pltpu.VMEM((CH,), jnp.int32),
            pltpu.VMEM((CH, C), jnp.float32),
            pltpu.SemaphoreType.DMA,
        ],
        interpret=interpret,
        **_MESH,
    )
    prop_kernel = pl.kernel(
        _prop_body,
        out_type=jax.ShapeDtypeStruct((NP, C), jnp.float32),
        scratch_types=[
            pltpu.VMEM_SHARED((NP, C), jnp.float32),
            pltpu.VMEM_SHARED((NP, C), jnp.float32),
            pltpu.VMEM((NCHUNK, CH), jnp.int32),
            pltpu.VMEM((NCHUNK, CH), jnp.int32),
            pltpu.VMEM((CH, C), jnp.float32),
            pltpu.VMEM((CH, C), jnp.float32),
            pltpu.VMEM((RPT, C), jnp.float32),
            pltpu.SemaphoreType.DMA,
            pltpu.SemaphoreType.DMA,
        ],
        interpret=interpret,
        **_MESH,
    )
    return degree_kernel, prop_kernel


_degree_kernel, _prop_kernel = _build()


def kernel(local_preds, edge_index, W1, W2):
    npad = EP - E
    row = jnp.concatenate(
        [edge_index[0], N + (jnp.arange(npad, dtype=jnp.int32) % (NP - N))])
    col = jnp.concatenate([edge_index[1], jnp.zeros(npad, dtype=jnp.int32)])
    x = jnp.pad(local_preds, ((0, NP - N), (0, 0)))
    ones_chunk = jnp.ones((CH, C), dtype=jnp.float32)

    deg_b = _degree_kernel(row, ones_chunk)
    q0, sa, sb, sal, sbl = _dense_stage(x, W1, W2, deg_b)

    col2 = col.reshape(NS * NCHUNK, CH)
    row2 = row.reshape(NS * NCHUNK, CH)
    preds = _prop_kernel(q0, col2, row2, sa, sb, sal, sbl)
    return preds[:N]


# speedup vs baseline: 59.2498x; 1.2272x over previous
import jax
import jax.numpy as jnp
from jax import lax
from jax.experimental import pallas as pl
from jax.experimental.pallas import tpu as pltpu
from jax.experimental.pallas import tpu_sc as plsc

N = 10000
E = 320000
IN_FEATS = 128
N_HIDDEN = 64
C = 16
ALPHA = 0.1
NITER = 10

NS = 16
NP = 10240
RPT = NP // NS
EP = 327680
EPT = EP // NS
CH = 1024
NCHUNK = EPT // CH

HC = C // 2
NPH = NP // 2
RPT2 = NPH // NS

_MESH1 = dict(
    mesh=plsc.VectorSubcoreMesh(
        core_axis_name="c", subcore_axis_name="s", num_cores=1, num_subcores=NS
    ),
    compiler_params=pltpu.CompilerParams(
        use_tc_tiling_on_sc=False, needs_layout_passes=False),
)
_MESH2 = dict(
    mesh=plsc.VectorSubcoreMesh(
        core_axis_name="c", subcore_axis_name="s", num_cores=2, num_subcores=NS
    ),
    compiler_params=pltpu.CompilerParams(
        use_tc_tiling_on_sc=False, needs_layout_passes=False),
)


def _degree_body(row_hbm, ones_hbm, deg_out, acc_sh, idx_v, ones_v, sem):
    w = lax.axis_index("s")
    base_r = pl.multiple_of(w * RPT, RPT)
    pltpu.sync_copy(ones_hbm, ones_v)
    pltpu.sync_copy(ones_v.at[pl.ds(0, RPT)], acc_sh.at[pl.ds(base_r, RPT)])
    plsc.subcore_barrier()

    def chunk(j, carry):
        e0 = pl.multiple_of(w * EPT + j * CH, CH)
        pltpu.sync_copy(row_hbm.at[pl.ds(e0, CH)], idx_v)
        pltpu.sync_copy(ones_v, acc_sh.at[idx_v], add=True)
        return carry

    lax.fori_loop(0, NCHUNK, chunk, 0)
    plsc.subcore_barrier()
    pltpu.sync_copy(acc_sh.at[pl.ds(base_r, RPT)], ones_v.at[pl.ds(0, RPT)])
    pltpu.sync_copy(ones_v.at[pl.ds(0, RPT)], deg_out.at[pl.ds(base_r, RPT)])


_BLK = 1024


def _dense_body(x_ref, w1_ref, w2_ref, deg_ref, q0_ref, sa_ref, sb_ref,
                sal_ref, sbl_ref):
    h = jnp.tanh(jnp.dot(x_ref[...], w1_ref[...],
                         preferred_element_type=jnp.float32))
    loc = jnp.dot(h, w2_ref[...], preferred_element_type=jnp.float32)
    dinv = lax.rsqrt(deg_ref[...])
    q0 = dinv * loc
    q0_ref[...] = q0
    sa_ref[...] = 0.9 * dinv * dinv
    sb_ref[...] = ALPHA * q0
    sal_ref[...] = 0.9 * dinv
    sbl_ref[...] = ALPHA * loc


def _dense_stage(x, w1, w2, deg_b):
    outs = [jax.ShapeDtypeStruct((NP, C), jnp.float32)] * 5
    return pl.pallas_call(
        _dense_body,
        grid=(NP // _BLK,),
        in_specs=[
            pl.BlockSpec((_BLK, IN_FEATS), lambda i: (i, 0)),
            pl.BlockSpec((IN_FEATS, N_HIDDEN), lambda i: (0, 0)),
            pl.BlockSpec((N_HIDDEN, C), lambda i: (0, 0)),
            pl.BlockSpec((_BLK, C), lambda i: (i, 0)),
        ],
        out_specs=[pl.BlockSpec((_BLK, C), lambda i: (i, 0))] * 5,
        out_shape=outs,
    )(x, w1, w2, deg_b)


def _prop_body(q0_hbm, col2_hbm, row2_hbm, sa_hbm, sb_hbm, sal_hbm, sbl_hbm,
               out_hbm,
               q_sh, acc_sh, cidx_a, ridx_a, msg0, msg1, a_v, b_v, c_v,
               sem0, sem1):
    cix = lax.axis_index("c")
    w = lax.axis_index("s")
    rsl = pl.ds(pl.multiple_of(w * RPT, RPT), RPT)
    pltpu.sync_copy(col2_hbm.at[pl.ds(w * NCHUNK, NCHUNK)], cidx_a)
    pltpu.sync_copy(row2_hbm.at[pl.ds(w * NCHUNK, NCHUNK)], ridx_a)
    bufs = (q_sh, acc_sh)
    pltpu.sync_copy(q0_hbm.at[cix, rsl], q_sh.at[rsl])
    pltpu.sync_copy(q0_hbm.at[cix, rsl], acc_sh.at[rsl])
    plsc.subcore_barrier()
    lane = lax.iota(jnp.int32, 16)
    r0 = lax.shift_right_logical(lane, 3)
    c16 = lax.bitwise_and(lane, 7)

    msgs = (msg0, msg1)
    sems = (sem0, sem1)

    for t in range(NITER):
        S = bufs[t % 2]
        ACC = bufs[(t + 1) % 2]
        sa = sa_hbm if t < NITER - 1 else sal_hbm
        sb = sb_hbm if t < NITER - 1 else sbl_hbm

        cps = [None, None]
        cps[0] = pltpu.async_copy(S.at[cidx_a.at[0]], msg0, sem0)
        for j in range(NCHUNK):
            b = j % 2
            cps[b].wait()
            if j + 1 < NCHUNK:
                nb = (j + 1) % 2
                cps[nb] = pltpu.async_copy(
                    S.at[cidx_a.at[j + 1]], msgs[nb], sems[nb])
            pltpu.sync_copy(msgs[b], ACC.at[ridx_a.at[j]], add=True)
        plsc.subcore_barrier()

        pltpu.sync_copy(ACC.at[rsl], a_v)
        pltpu.sync_copy(sa.at[rsl], b_v)
        pltpu.sync_copy(sb.at[cix, rsl], c_v)

        def row(i, carry):
            ri = r0 + 2 * i
            a = plsc.load_gather(a_v, [ri, c16])
            bb = plsc.load_gather(b_v, [ri, c16])
            cc = plsc.load_gather(c_v, [ri, c16])
            plsc.store_scatter(a_v, [ri, c16], a * bb + cc)
            return carry

        lax.fori_loop(0, RPT // 2, row, 0)
        if t < NITER - 1:
            pltpu.sync_copy(a_v, ACC.at[rsl])
            pltpu.sync_copy(a_v, S.at[rsl])
        else:
            pltpu.sync_copy(a_v, out_hbm.at[cix, rsl])
        plsc.subcore_barrier()


def _build(interpret=False):
    degree_kernel = pl.kernel(
        _degree_body,
        out_type=jax.ShapeDtypeStruct((NP, C), jnp.float32),
        scratch_types=[
            pltpu.VMEM_SHARED((NP, C), jnp.float32),
            pltpu.VMEM((CH,), jnp.int32),
            pltpu.VMEM((CH, C), jnp.float32),
            pltpu.SemaphoreType.DMA,
        ],
        interpret=interpret,
        **_MESH1,
    )
    prop_kernel = pl.kernel(
        _prop_body,
        out_type=jax.ShapeDtypeStruct((2, NP, HC), jnp.float32),
        scratch_types=[
            pltpu.VMEM_SHARED((NP, HC), jnp.float32),
            pltpu.VMEM_SHARED((NP, HC), jnp.float32),
            pltpu.VMEM((NCHUNK, CH), jnp.int32),
            pltpu.VMEM((NCHUNK, CH), jnp.int32),
            pltpu.VMEM((CH, HC), jnp.float32),
            pltpu.VMEM((CH, HC), jnp.float32),
            pltpu.VMEM((RPT, HC), jnp.float32),
            pltpu.VMEM((RPT, HC), jnp.float32),
            pltpu.VMEM((RPT, HC), jnp.float32),
            pltpu.SemaphoreType.DMA,
            pltpu.SemaphoreType.DMA,
        ],
        interpret=interpret,
        **_MESH2,
    )
    return degree_kernel, prop_kernel


_degree_kernel, _prop_kernel = _build()


def kernel(local_preds, edge_index, W1, W2):
    npad = EP - E
    row = jnp.concatenate(
        [edge_index[0], N + (jnp.arange(npad, dtype=jnp.int32) % (NP - N))])
    col = jnp.concatenate([edge_index[1], jnp.zeros(npad, dtype=jnp.int32)])
    x = jnp.pad(local_preds, ((0, NP - N), (0, 0)))
    ones_chunk = jnp.ones((CH, C), dtype=jnp.float32)

    deg_b = _degree_kernel(row, ones_chunk)
    q0, sa, sb, sal, sbl = _dense_stage(x, W1, W2, deg_b)

    col2 = col.reshape(NS * NCHUNK, CH)
    row2 = row.reshape(NS * NCHUNK, CH)

    def halves(arr):
        return jnp.stack([arr[:, :HC], arr[:, HC:]])

    out = _prop_kernel(halves(q0), col2, row2,
                       sa[:, :HC], halves(sb),
                       sal[:, :HC], halves(sbl))
    preds = jnp.concatenate([out[0], out[1]], axis=1)
    return preds[:N]


# speedup vs baseline: 63.8230x; 1.0772x over previous
import jax
import jax.numpy as jnp
from jax import lax
from jax.experimental import pallas as pl
from jax.experimental.pallas import tpu as pltpu
from jax.experimental.pallas import tpu_sc as plsc

N = 10000
E = 320000
IN_FEATS = 128
N_HIDDEN = 64
C = 16
ALPHA = 0.1
NITER = 10

NS = 16
NP = 10240
RPT = NP // NS
EP = 327680
EPT = EP // NS
CH = 1024
NCHUNK = EPT // CH

HC = C // 2
NPH = NP // 2
RPT2 = NPH // NS

_MESH1 = dict(
    mesh=plsc.VectorSubcoreMesh(
        core_axis_name="c", subcore_axis_name="s", num_cores=1, num_subcores=NS
    ),
    compiler_params=pltpu.CompilerParams(
        use_tc_tiling_on_sc=False, needs_layout_passes=False),
)
_MESH2 = dict(
    mesh=plsc.VectorSubcoreMesh(
        core_axis_name="c", subcore_axis_name="s", num_cores=2, num_subcores=NS
    ),
    compiler_params=pltpu.CompilerParams(
        use_tc_tiling_on_sc=False, needs_layout_passes=False),
)


def _degree_body(row_hbm, ones_hbm, deg_out, acc_sh, idx_v, ones_v, sem):
    w = lax.axis_index("s")
    base_r = pl.multiple_of(w * RPT, RPT)
    pltpu.sync_copy(ones_hbm, ones_v)
    pltpu.sync_copy(ones_v.at[pl.ds(0, RPT)], acc_sh.at[pl.ds(base_r, RPT)])
    plsc.subcore_barrier()

    def chunk(j, carry):
        e0 = pl.multiple_of(w * EPT + j * CH, CH)
        pltpu.sync_copy(row_hbm.at[pl.ds(e0, CH)], idx_v)
        pltpu.sync_copy(ones_v, acc_sh.at[idx_v], add=True)
        return carry

    lax.fori_loop(0, NCHUNK, chunk, 0)
    plsc.subcore_barrier()
    pltpu.sync_copy(acc_sh.at[pl.ds(base_r, RPT)], ones_v.at[pl.ds(0, RPT)])
    pltpu.sync_copy(ones_v.at[pl.ds(0, RPT)], deg_out.at[pl.ds(base_r, RPT)])


_BLK = 1024


def _dense_body(x_ref, w1_ref, w2_ref, deg_ref, q0_ref, sa_ref, sb_ref,
                sal_ref, sbl_ref):
    h = jnp.tanh(jnp.dot(x_ref[...], w1_ref[...],
                         preferred_element_type=jnp.float32))
    loc = jnp.dot(h, w2_ref[...], preferred_element_type=jnp.float32)
    dinv = lax.rsqrt(deg_ref[...])
    q0 = dinv * loc
    q0_ref[...] = q0
    sa_ref[...] = 0.9 * dinv * dinv
    sb_ref[...] = ALPHA * q0
    sal_ref[...] = 0.9 * dinv
    sbl_ref[...] = ALPHA * loc


def _dense_stage(x, w1, w2, deg_b):
    outs = [jax.ShapeDtypeStruct((NP, C), jnp.float32)] * 5
    return pl.pallas_call(
        _dense_body,
        grid=(NP // _BLK,),
        in_specs=[
            pl.BlockSpec((_BLK, IN_FEATS), lambda i: (i, 0)),
            pl.BlockSpec((IN_FEATS, N_HIDDEN), lambda i: (0, 0)),
            pl.BlockSpec((N_HIDDEN, C), lambda i: (0, 0)),
            pl.BlockSpec((_BLK, C), lambda i: (i, 0)),
        ],
        out_specs=[pl.BlockSpec((_BLK, C), lambda i: (i, 0))] * 5,
        out_shape=outs,
    )(x, w1, w2, deg_b)


NBUF = 4


def _prop_body(q0_hbm, col2_hbm, row2_hbm, sa_hbm, sb_hbm, sal_hbm, sbl_hbm,
               out_hbm,
               q_sh, acc_sh, cidx_a, ridx_a, msg0, msg1, msg2, msg3,
               a_v, b_v, c_v, gs0, gs1, gs2, gs3, ss0, ss1, ss2, ss3):
    cix = lax.axis_index("c")
    w = lax.axis_index("s")
    rsl = pl.ds(pl.multiple_of(w * RPT, RPT), RPT)
    pltpu.sync_copy(col2_hbm.at[pl.ds(w * NCHUNK, NCHUNK)], cidx_a)
    pltpu.sync_copy(row2_hbm.at[pl.ds(w * NCHUNK, NCHUNK)], ridx_a)
    bufs = (q_sh, acc_sh)
    pltpu.sync_copy(q0_hbm.at[cix, rsl], q_sh.at[rsl])
    pltpu.sync_copy(q0_hbm.at[cix, rsl], acc_sh.at[rsl])
    plsc.subcore_barrier()
    lane = lax.iota(jnp.int32, 16)
    r0 = lax.shift_right_logical(lane, 3)
    c16 = lax.bitwise_and(lane, 7)

    msgs = (msg0, msg1, msg2, msg3)
    gsems = (gs0, gs1, gs2, gs3)
    ssems = (ss0, ss1, ss2, ss3)

    for t in range(NITER):
        S = bufs[t % 2]
        ACC = bufs[(t + 1) % 2]
        sa = sa_hbm if t < NITER - 1 else sal_hbm
        sb = sb_hbm if t < NITER - 1 else sbl_hbm

        gcp = [None] * NCHUNK
        scp = [None] * NCHUNK
        gcp[0] = pltpu.async_copy(S.at[cidx_a.at[0]], msgs[0], gsems[0])
        if NCHUNK > 1:
            gcp[1] = pltpu.async_copy(S.at[cidx_a.at[1]], msgs[1], gsems[1])
        for j in range(NCHUNK):
            b = j % NBUF
            gcp[j].wait()
            scp[j] = pltpu.async_copy(
                msgs[b], ACC.at[ridx_a.at[j]], ssems[b], add=True)
            nxt = j + 2
            if nxt < NCHUNK:
                nb = nxt % NBUF
                if nxt >= NBUF:
                    scp[nxt - NBUF].wait()
                gcp[nxt] = pltpu.async_copy(
                    S.at[cidx_a.at[nxt]], msgs[nb], gsems[nb])
        for j in range(max(0, NCHUNK - NBUF), NCHUNK):
            scp[j].wait()
        plsc.subcore_barrier()

        pltpu.sync_copy(ACC.at[rsl], a_v)
        pltpu.sync_copy(sa.at[rsl], b_v)
        pltpu.sync_copy(sb.at[cix, rsl], c_v)

        def row(i, carry):
            ri = r0 + 2 * i
            a = plsc.load_gather(a_v, [ri, c16])
            bb = plsc.load_gather(b_v, [ri, c16])
            cc = plsc.load_gather(c_v, [ri, c16])
            plsc.store_scatter(a_v, [ri, c16], a * bb + cc)
            return carry

        lax.fori_loop(0, RPT // 2, row, 0)
        if t < NITER - 1:
            pltpu.sync_copy(a_v, ACC.at[rsl])
            pltpu.sync_copy(a_v, S.at[rsl])
        else:
            pltpu.sync_copy(a_v, out_hbm.at[cix, rsl])
        plsc.subcore_barrier()


def _build(interpret=False):
    degree_kernel = pl.kernel(
        _degree_body,
        out_type=jax.ShapeDtypeStruct((NP, C), jnp.float32),
        scratch_types=[
            pltpu.VMEM_SHARED((NP, C), jnp.float32),
            pltpu.VMEM((CH,), jnp.int32),
            pltpu.VMEM((CH, C), jnp.float32),
            pltpu.SemaphoreType.DMA,
        ],
        interpret=interpret,
        **_MESH1,
    )
    prop_kernel = pl.kernel(
        _prop_body,
        out_type=jax.ShapeDtypeStruct((2, NP, HC), jnp.float32),
        scratch_types=[
            pltpu.VMEM_SHARED((NP, HC), jnp.float32),
            pltpu.VMEM_SHARED((NP, HC), jnp.float32),
            pltpu.VMEM((NCHUNK, CH), jnp.int32),
            pltpu.VMEM((NCHUNK, CH), jnp.int32),
            pltpu.VMEM((CH, HC), jnp.float32),
            pltpu.VMEM((CH, HC), jnp.float32),
            pltpu.VMEM((CH, HC), jnp.float32),
            pltpu.VMEM((CH, HC), jnp.float32),
            pltpu.VMEM((RPT, HC), jnp.float32),
            pltpu.VMEM((RPT, HC), jnp.float32),
            pltpu.VMEM((RPT, HC), jnp.float32),
        ] + [pltpu.SemaphoreType.DMA] * 8,
        interpret=interpret,
        **_MESH2,
    )
    return degree_kernel, prop_kernel


_degree_kernel, _prop_kernel = _build()


def kernel(local_preds, edge_index, W1, W2):
    npad = EP - E
    row = jnp.concatenate(
        [edge_index[0], N + (jnp.arange(npad, dtype=jnp.int32) % (NP - N))])
    col = jnp.concatenate([edge_index[1], jnp.zeros(npad, dtype=jnp.int32)])
    x = jnp.pad(local_preds, ((0, NP - N), (0, 0)))
    ones_chunk = jnp.ones((CH, C), dtype=jnp.float32)

    deg_b = _degree_kernel(row, ones_chunk)
    q0, sa, sb, sal, sbl = _dense_stage(x, W1, W2, deg_b)

    col2 = col.reshape(NS * NCHUNK, CH)
    row2 = row.reshape(NS * NCHUNK, CH)

    def halves(arr):
        return jnp.stack([arr[:, :HC], arr[:, HC:]])

    out = _prop_kernel(halves(q0), col2, row2,
                       sa[:, :HC], halves(sb),
                       sal[:, :HC], halves(sbl))
    preds = jnp.concatenate([out[0], out[1]], axis=1)
    return preds[:N]


# speedup vs baseline: 82.5434x; 1.2933x over previous
import jax
import jax.numpy as jnp
from jax import lax
from jax.experimental import pallas as pl
from jax.experimental.pallas import tpu as pltpu
from jax.experimental.pallas import tpu_sc as plsc

N = 10000
E = 320000
IN_FEATS = 128
N_HIDDEN = 64
C = 16
ALPHA = 0.1
NITER = 10

NS = 16
NP = 10240
RPT = NP // NS
EP = 327680
EPT = EP // NS
CH = 1024
NCHUNK = EPT // CH

HC = C // 2
NPH = NP // 2
RPT2 = NPH // NS

_MESH1 = dict(
    mesh=plsc.VectorSubcoreMesh(
        core_axis_name="c", subcore_axis_name="s", num_cores=1, num_subcores=NS
    ),
    compiler_params=pltpu.CompilerParams(
        use_tc_tiling_on_sc=False, needs_layout_passes=False),
)
_MESH2 = dict(
    mesh=plsc.VectorSubcoreMesh(
        core_axis_name="c", subcore_axis_name="s", num_cores=2, num_subcores=NS
    ),
    compiler_params=pltpu.CompilerParams(
        use_tc_tiling_on_sc=False, needs_layout_passes=False),
)


def _degree_body(row2_hbm, ones_hbm, zeros_hbm, deg_out,
                 acc_sh, cidx_a, ones_v, sem):
    cix = lax.axis_index("c")
    w = lax.axis_index("s")
    rsl = pl.ds(pl.multiple_of(w * RPT, RPT), RPT)
    pltpu.sync_copy(row2_hbm.at[pl.ds(w * NCHUNK, NCHUNK)], cidx_a)
    pltpu.sync_copy(ones_hbm, ones_v)
    pltpu.sync_copy(zeros_hbm, acc_sh.at[rsl])
    plsc.subcore_barrier()
    cps = [pltpu.async_copy(ones_v, acc_sh.at[cidx_a.at[j]], sem, add=True)
           for j in range(NCHUNK)]
    for cp in cps:
        cp.wait()
    plsc.subcore_barrier()
    pltpu.sync_copy(acc_sh.at[rsl], deg_out.at[cix, rsl])


_BLK = 1024


def _dense_body(x_ref, w1_ref, w2_ref, loc_ref):
    h = jnp.tanh(jnp.dot(x_ref[...], w1_ref[...],
                         preferred_element_type=jnp.float32))
    loc_ref[...] = jnp.dot(h, w2_ref[...], preferred_element_type=jnp.float32)


def _dense_stage(x, w1, w2):
    return pl.pallas_call(
        _dense_body,
        grid=(NP // _BLK,),
        in_specs=[
            pl.BlockSpec((_BLK, IN_FEATS), lambda i: (i, 0)),
            pl.BlockSpec((IN_FEATS, N_HIDDEN), lambda i: (0, 0)),
            pl.BlockSpec((N_HIDDEN, C), lambda i: (0, 0)),
        ],
        out_specs=pl.BlockSpec((_BLK, C), lambda i: (i, 0)),
        out_shape=jax.ShapeDtypeStruct((NP, C), jnp.float32),
    )(x, w1, w2)


NBUF = 4


def _prop_body(deg2_hbm, loc2_hbm, col2_hbm, row2_hbm,
               out_hbm,
               q_sh, acc_sh, cidx_a, ridx_a, msg0, msg1, msg2, msg3,
               a_v, b_v, sa_v, sb_v, sal_v, sbl_v,
               gs0, gs1, gs2, gs3, ss0, ss1, ss2, ss3):
    cix = lax.axis_index("c")
    w = lax.axis_index("s")
    rsl = pl.ds(pl.multiple_of(w * RPT, RPT), RPT)
    pltpu.sync_copy(col2_hbm.at[pl.ds(w * NCHUNK, NCHUNK)], cidx_a)
    pltpu.sync_copy(row2_hbm.at[pl.ds(w * NCHUNK, NCHUNK)], ridx_a)
    lane = lax.iota(jnp.int32, 16)
    r0 = lax.shift_right_logical(lane, 3)
    c16 = lax.bitwise_and(lane, 7)

    pltpu.sync_copy(deg2_hbm.at[cix, rsl], a_v)
    pltpu.sync_copy(loc2_hbm.at[cix, rsl], b_v)

    def prep(i, carry):
        ri = r0 + 2 * i
        d = plsc.load_gather(a_v, [ri, c16]) + 1.0
        lo = plsc.load_gather(b_v, [ri, c16])
        ii = 0x5F3759DF - lax.shift_right_logical(
            plsc.bitcast(d, jnp.int32), 1)
        y = plsc.bitcast(ii, jnp.float32)
        for _ in range(4):
            y = y * (1.5 - 0.5 * d * y * y)
        q0 = y * lo
        plsc.store_scatter(sa_v, [ri, c16], 0.9 * y * y)
        plsc.store_scatter(sal_v, [ri, c16], 0.9 * y)
        plsc.store_scatter(sb_v, [ri, c16], ALPHA * q0)
        plsc.store_scatter(sbl_v, [ri, c16], ALPHA * lo)
        plsc.store_scatter(a_v, [ri, c16], q0)
        return carry

    lax.fori_loop(0, RPT // 2, prep, 0)
    bufs = (q_sh, acc_sh)
    pltpu.sync_copy(a_v, q_sh.at[rsl])
    pltpu.sync_copy(a_v, acc_sh.at[rsl])
    plsc.subcore_barrier()

    msgs = (msg0, msg1, msg2, msg3)
    gsems = (gs0, gs1, gs2, gs3)
    ssems = (ss0, ss1, ss2, ss3)

    for t in range(NITER):
        S = bufs[t % 2]
        ACC = bufs[(t + 1) % 2]
        s1_v = sa_v if t < NITER - 1 else sal_v
        s2_v = sb_v if t < NITER - 1 else sbl_v

        gcp = [None] * NCHUNK
        scp = [None] * NCHUNK
        gcp[0] = pltpu.async_copy(S.at[cidx_a.at[0]], msgs[0], gsems[0])
        if NCHUNK > 1:
            gcp[1] = pltpu.async_copy(S.at[cidx_a.at[1]], msgs[1], gsems[1])
        for j in range(NCHUNK):
            b = j % NBUF
            gcp[j].wait()
            scp[j] = pltpu.async_copy(
                msgs[b], ACC.at[ridx_a.at[j]], ssems[b], add=True)
            nxt = j + 2
            if nxt < NCHUNK:
                nb = nxt % NBUF
                if nxt >= NBUF:
                    scp[nxt - NBUF].wait()
                gcp[nxt] = pltpu.async_copy(
                    S.at[cidx_a.at[nxt]], msgs[nb], gsems[nb])
        for j in range(max(0, NCHUNK - NBUF), NCHUNK):
            scp[j].wait()
        plsc.subcore_barrier()

        pltpu.sync_copy(ACC.at[rsl], a_v)

        def row(i, carry):
            ri = r0 + 2 * i
            a = plsc.load_gather(a_v, [ri, c16])
            bb = plsc.load_gather(s1_v, [ri, c16])
            cc = plsc.load_gather(s2_v, [ri, c16])
            plsc.store_scatter(a_v, [ri, c16], a * bb + cc)
            return carry

        lax.fori_loop(0, RPT // 2, row, 0)
        if t < NITER - 1:
            pltpu.sync_copy(a_v, ACC.at[rsl])
            pltpu.sync_copy(a_v, S.at[rsl])
        else:
            pltpu.sync_copy(a_v, out_hbm.at[cix, rsl])
        plsc.subcore_barrier()


def _build(interpret=False):
    degree_kernel = pl.kernel(
        _degree_body,
        out_type=jax.ShapeDtypeStruct((2, NP, HC), jnp.float32),
        scratch_types=[
            pltpu.VMEM_SHARED((NP, HC), jnp.float32),
            pltpu.VMEM((NCHUNK, CH), jnp.int32),
            pltpu.VMEM((CH, HC), jnp.float32),
            pltpu.SemaphoreType.DMA,
        ],
        interpret=interpret,
        **_MESH2,
    )
    prop_kernel = pl.kernel(
        _prop_body,
        out_type=jax.ShapeDtypeStruct((2, NP, HC), jnp.float32),
        scratch_types=[
            pltpu.VMEM_SHARED((NP, HC), jnp.float32),
            pltpu.VMEM_SHARED((NP, HC), jnp.float32),
            pltpu.VMEM((NCHUNK, CH), jnp.int32),
            pltpu.VMEM((NCHUNK, CH), jnp.int32),
            pltpu.VMEM((CH, HC), jnp.float32),
            pltpu.VMEM((CH, HC), jnp.float32),
            pltpu.VMEM((CH, HC), jnp.float32),
            pltpu.VMEM((CH, HC), jnp.float32),
            pltpu.VMEM((RPT, HC), jnp.float32),
            pltpu.VMEM((RPT, HC), jnp.float32),
            pltpu.VMEM((RPT, HC), jnp.float32),
            pltpu.VMEM((RPT, HC), jnp.float32),
            pltpu.VMEM((RPT, HC), jnp.float32),
            pltpu.VMEM((RPT, HC), jnp.float32),
        ] + [pltpu.SemaphoreType.DMA] * 8,
        interpret=interpret,
        **_MESH2,
    )
    return degree_kernel, prop_kernel


_degree_kernel, _prop_kernel = _build()


def kernel(local_preds, edge_index, W1, W2):
    npad = EP - E
    row = jnp.concatenate(
        [edge_index[0], N + (jnp.arange(npad, dtype=jnp.int32) % (NP - N))])
    col = jnp.concatenate([edge_index[1], jnp.zeros(npad, dtype=jnp.int32)])
    x = jnp.pad(local_preds, ((0, NP - N), (0, 0)))
    ones8 = jnp.ones((CH, HC), dtype=jnp.float32)
    zeros8 = jnp.zeros((RPT, HC), dtype=jnp.float32)

    col2 = col.reshape(NS * NCHUNK, CH)
    row2 = row.reshape(NS * NCHUNK, CH)

    deg2 = _degree_kernel(row2, ones8, zeros8)
    loc = _dense_stage(x, W1, W2)
    loc2 = jnp.stack([loc[:, :HC], loc[:, HC:]])

    out = _prop_kernel(deg2, loc2, col2, row2)
    preds = jnp.concatenate([out[0], out[1]], axis=1)
    return preds[:N]
